# baseline (speedup 1.0000x reference)
import functools

import jax
import jax.numpy as jnp
from jax import lax
from jax.experimental import pallas as pl
from jax.experimental.pallas import tpu as pltpu
from jax.experimental.pallas import tpu_sc as plsc

VOC = 100000
EMB = 64
SCALE = 8.0

NC = 2
NS = 16
NW = NC * NS

B = 4096 * 200
BPW = B // NW
CHUNK = 128
NCHUNK = BPW // CHUNK
NBUF = 8
NOUT = NCHUNK // NBUF

TF = VOC * EMB
FPW = TF // NW
SCHUNK = 40000
NSCHUNK = FPW // SCHUNK

_mesh = plsc.VectorSubcoreMesh(core_axis_name="c", subcore_axis_name="s")
_params = pltpu.CompilerParams(use_tc_tiling_on_sc=False)


@functools.partial(
    pl.kernel,
    mesh=_mesh,
    out_type=jax.ShapeDtypeStruct((TF,), jnp.float32),
    scratch_types=[
        pltpu.VMEM((2, SCHUNK), jnp.float32),
        [pltpu.SemaphoreType.DMA] * 2,
        [pltpu.SemaphoreType.DMA] * 2,
    ],
    compiler_params=_params,
)
def _scale_table(table_hbm, out_hbm, buf, isems, osems):
    wid = lax.axis_index("s") * NC + lax.axis_index("c")
    base = wid * FPW

    def start_in(g, b):
        pltpu.async_copy(
            table_hbm.at[pl.ds(base + g * SCHUNK, SCHUNK)], buf.at[b], isems[b]
        )

    def wait_in(g, b):
        pltpu.make_async_copy(
            table_hbm.at[pl.ds(base + g * SCHUNK, SCHUNK)], buf.at[b], isems[b]
        ).wait()

    def scale_buf(b):
        def body(i, _):
            for k in range(8):
                sl = pl.ds((i * 8 + k) * 16, 16)
                buf[b, sl] = buf[b, sl] * SCALE
            return 0

        lax.fori_loop(0, SCHUNK // 128, body, 0)

    def store_out(g, b):
        cp = pltpu.make_async_copy(
            buf.at[b], out_hbm.at[pl.ds(base + g * SCHUNK, SCHUNK)], osems[b]
        )
        cp.start()
        cp.wait()

    start_in(0, 0)
    for g in range(NSCHUNK):
        b = g % 2
        wait_in(g, b)
        if g + 1 < NSCHUNK:
            start_in(g + 1, (g + 1) % 2)
        scale_buf(b)
        store_out(g, b)


@functools.partial(
    pl.kernel,
    mesh=_mesh,
    out_type=jax.ShapeDtypeStruct((B, EMB), jnp.float32),
    scratch_types=[
        pltpu.VMEM((BPW,), jnp.int32),
        pltpu.VMEM((NBUF, CHUNK, EMB), jnp.float32),
        [pltpu.SemaphoreType.DMA] * NBUF,
        [pltpu.SemaphoreType.DMA] * NBUF,
    ],
    compiler_params=_params,
)
def _emb_lookup(tokens_hbm, table_hbm, out_hbm, idx_v, rows_v, gsems, ssems):
    wid = lax.axis_index("s") * NC + lax.axis_index("c")
    base = wid * BPW
    pltpu.sync_copy(tokens_hbm.at[pl.ds(base, BPW)], idx_v)

    def start_gather(g, b):
        pltpu.async_copy(
            table_hbm.at[idx_v.at[pl.ds(g * CHUNK, CHUNK)]],
            rows_v.at[b],
            gsems[b],
        )

    def wait_gather(g, b):
        pltpu.make_async_copy(
            table_hbm.at[idx_v.at[pl.ds(g * CHUNK, CHUNK)]],
            rows_v.at[b],
            gsems[b],
        ).wait()

    def scatter(g, b):
        cp = pltpu.make_async_copy(
            rows_v.at[b], out_hbm.at[pl.ds(base + g * CHUNK, CHUNK)], ssems[b]
        )
        cp.start()
        cp.wait()

    for b in range(NBUF):
        start_gather(b, b)

    def outer(i, _):
        for b in range(NBUF):
            g = i * NBUF + b
            wait_gather(g, b)
            scatter(g, b)
            start_gather(g + NBUF, b)
        return 0

    lax.fori_loop(0, NOUT - 1, outer, 0)

    for b in range(NBUF):
        g = (NOUT - 1) * NBUF + b
        wait_gather(g, b)
        scatter(g, b)


def kernel(tokens, table):
    scaled = _scale_table(table.reshape(-1)).reshape(VOC, EMB)
    out = _emb_lookup(tokens.reshape(-1), scaled)
    return out.reshape(tokens.shape + (EMB,))


# speedup vs baseline: 1.0019x; 1.0019x over previous
import functools

import jax
import jax.numpy as jnp
from jax import lax
from jax.experimental import pallas as pl
from jax.experimental.pallas import tpu as pltpu
from jax.experimental.pallas import tpu_sc as plsc

VOC = 100000
EMB = 64
SCALE = 8.0

NC = 2
NS = 16
NW = NC * NS

B = 4096 * 200
BPW = B // NW
CHUNK = 128
NCHUNK = BPW // CHUNK
NBUF = 8
NOUT = NCHUNK // NBUF

TF = VOC * EMB
FPW = TF // NW
SCHUNK = 40000
NSCHUNK = FPW // SCHUNK

_mesh = plsc.VectorSubcoreMesh(core_axis_name="c", subcore_axis_name="s")
_params = pltpu.CompilerParams(use_tc_tiling_on_sc=False)


@functools.partial(
    pl.kernel,
    mesh=_mesh,
    out_type=jax.ShapeDtypeStruct((TF,), jnp.float32),
    scratch_types=[
        pltpu.VMEM((2, SCHUNK), jnp.float32),
        [pltpu.SemaphoreType.DMA] * 2,
        [pltpu.SemaphoreType.DMA] * 2,
    ],
    compiler_params=_params,
)
def _scale_table(table_hbm, out_hbm, buf, isems, osems):
    wid = lax.axis_index("s") * NC + lax.axis_index("c")
    base = wid * FPW

    def start_in(g, b):
        pltpu.async_copy(
            table_hbm.at[pl.ds(base + g * SCHUNK, SCHUNK)], buf.at[b], isems[b]
        )

    def wait_in(g, b):
        pltpu.make_async_copy(
            table_hbm.at[pl.ds(base + g * SCHUNK, SCHUNK)], buf.at[b], isems[b]
        ).wait()

    def scale_buf(b):
        def body(i, _):
            for k in range(10):
                sl = pl.ds((i * 10 + k) * 16, 16)
                buf[b, sl] = buf[b, sl] * SCALE
            return 0

        lax.fori_loop(0, SCHUNK // 160, body, 0)

    def store_out(g, b):
        cp = pltpu.make_async_copy(
            buf.at[b], out_hbm.at[pl.ds(base + g * SCHUNK, SCHUNK)], osems[b]
        )
        cp.start()
        cp.wait()

    start_in(0, 0)
    for g in range(NSCHUNK):
        b = g % 2
        wait_in(g, b)
        if g + 1 < NSCHUNK:
            start_in(g + 1, (g + 1) % 2)
        scale_buf(b)
        store_out(g, b)


@functools.partial(
    pl.kernel,
    mesh=_mesh,
    out_type=jax.ShapeDtypeStruct((B, EMB), jnp.float32),
    scratch_types=[
        pltpu.VMEM((BPW,), jnp.int32),
        pltpu.VMEM((NBUF, CHUNK, EMB), jnp.float32),
        [pltpu.SemaphoreType.DMA] * NBUF,
        [pltpu.SemaphoreType.DMA] * NBUF,
    ],
    compiler_params=_params,
)
def _emb_lookup(tokens_hbm, table_hbm, out_hbm, idx_v, rows_v, gsems, ssems):
    wid = lax.axis_index("s") * NC + lax.axis_index("c")
    base = wid * BPW
    pltpu.sync_copy(tokens_hbm.at[pl.ds(base, BPW)], idx_v)

    def start_gather(g, b):
        pltpu.async_copy(
            table_hbm.at[idx_v.at[pl.ds(g * CHUNK, CHUNK)]],
            rows_v.at[b],
            gsems[b],
        )

    def wait_gather(g, b):
        pltpu.make_async_copy(
            table_hbm.at[idx_v.at[pl.ds(g * CHUNK, CHUNK)]],
            rows_v.at[b],
            gsems[b],
        ).wait()

    def start_scatter(g, b):
        pltpu.async_copy(
            rows_v.at[b], out_hbm.at[pl.ds(base + g * CHUNK, CHUNK)], ssems[b]
        )

    def wait_scatter(g, b):
        pltpu.make_async_copy(
            rows_v.at[b], out_hbm.at[pl.ds(base + g * CHUNK, CHUNK)], ssems[b]
        ).wait()

    D = 4

    for g in range(D):
        start_gather(g, g)

    for b in range(NBUF):
        wait_gather(b, b)
        start_scatter(b, b)
        if b >= NBUF - D:
            wait_scatter(b - (NBUF - D), (b + D) % NBUF)
        start_gather(b + D, (b + D) % NBUF)

    def outer(i, _):
        for b in range(NBUF):
            g = i * NBUF + b
            wait_gather(g, b)
            start_scatter(g, b)
            wait_scatter(g - (NBUF - D), (b + D) % NBUF)
            start_gather(g + D, (b + D) % NBUF)
        return 0

    lax.fori_loop(1, NOUT - 1, outer, 0)

    for b in range(NBUF):
        g = (NOUT - 1) * NBUF + b
        wait_gather(g, b)
        start_scatter(g, b)
        if b < D:
            wait_scatter(g - (NBUF - D), (b + D) % NBUF)
            start_gather(g + D, (b + D) % NBUF)

    for b in range(NBUF):
        wait_scatter((NOUT - 1) * NBUF + b, b)


def kernel(tokens, table):
    scaled = _scale_table(table.reshape(-1)).reshape(VOC, EMB)
    out = _emb_lookup(tokens.reshape(-1), scaled)
    return out.reshape(tokens.shape + (EMB,))


# speedup vs baseline: 1.0765x; 1.0744x over previous
import functools

import jax
import jax.numpy as jnp
from jax import lax
from jax.experimental import pallas as pl
from jax.experimental.pallas import tpu as pltpu
from jax.experimental.pallas import tpu_sc as plsc

VOC = 100000
EMB = 64
PAD = 128
SCALE = 8.0

NC = 2
NS = 16
NW = NC * NS

NB = 4096
NT = 200
RPW = NB // NW

SR = 400
NSC = VOC // SR

_mesh = plsc.VectorSubcoreMesh(core_axis_name="c", subcore_axis_name="s")
_params = pltpu.CompilerParams(use_tc_tiling_on_sc=True)


@functools.partial(
    pl.kernel,
    mesh=_mesh,
    out_type=jax.ShapeDtypeStruct((VOC, PAD), jnp.float32),
    scratch_types=[
        pltpu.VMEM((SR, EMB), jnp.float32),
        pltpu.VMEM((SR, PAD), jnp.float32),
        pltpu.SemaphoreType.DMA,
        pltpu.SemaphoreType.DMA,
    ],
    compiler_params=_params,
)
def _scale_table(table_hbm, out_hbm, in_v, img_v, isem, osem):
    wid = lax.axis_index("s") * NC + lax.axis_index("c")
    nmine = (NSC - wid + NW - 1) // NW

    def chunk(i, _):
        row0 = (wid + i * NW) * SR
        pltpu.async_copy(
            table_hbm.at[pl.ds(row0, SR), :], in_v, isem
        ).wait()

        def body(r, _):
            for j in range(EMB // 16):
                img_v[r, pl.ds(j * 16, 16)] = in_v[r, pl.ds(j * 16, 16)] * SCALE
            return 0

        lax.fori_loop(0, SR, body, 0)
        cp = pltpu.make_async_copy(img_v, out_hbm.at[pl.ds(row0, SR), :], osem)
        cp.start()
        cp.wait()
        return 0

    lax.fori_loop(0, nmine, chunk, 0)


NBUF = 2


@functools.partial(
    pl.kernel,
    mesh=_mesh,
    out_type=jax.ShapeDtypeStruct((NB, NT, EMB), jnp.float32),
    scratch_types=[
        pltpu.VMEM((RPW * NT,), jnp.int32),
        pltpu.VMEM((NBUF, NT, PAD), jnp.float32),
        pltpu.VMEM((NBUF, NT, EMB), jnp.float32),
        [pltpu.SemaphoreType.DMA] * NBUF,
        [pltpu.SemaphoreType.DMA] * NBUF,
    ],
    compiler_params=_params,
)
def _emb_lookup(tokens_hbm, table_hbm, out_hbm, idx_v, rows_v, comp_v,
                gsems, ssems):
    wid = lax.axis_index("s") * NC + lax.axis_index("c")
    base = wid * RPW * NT

    pltpu.sync_copy(tokens_hbm.at[pl.ds(base, RPW * NT)], idx_v)

    def start_gathers(r, b):
        pltpu.async_copy(
            table_hbm.at[idx_v.at[pl.ds(r * NT, 128)]],
            rows_v.at[b, pl.ds(0, 128)],
            gsems[b],
        )
        pltpu.async_copy(
            table_hbm.at[idx_v.at[pl.ds(r * NT + 128, NT - 128)]],
            rows_v.at[b, pl.ds(128, NT - 128)],
            gsems[b],
        )

    def wait_gathers(r, b):
        pltpu.make_async_copy(
            table_hbm.at[idx_v.at[pl.ds(r * NT, 128)]],
            rows_v.at[b, pl.ds(0, 128)],
            gsems[b],
        ).wait()
        pltpu.make_async_copy(
            table_hbm.at[idx_v.at[pl.ds(r * NT + 128, NT - 128)]],
            rows_v.at[b, pl.ds(128, NT - 128)],
            gsems[b],
        ).wait()

    def compact(b):
        def body(r, _):
            for j in range(EMB // 16):
                comp_v[b, r, pl.ds(j * 16, 16)] = rows_v[b, r, pl.ds(j * 16, 16)]
            return 0

        lax.fori_loop(0, NT, body, 0)

    def start_write(r, b):
        pltpu.async_copy(
            comp_v.at[b], out_hbm.at[wid * RPW + r], ssems[b]
        )

    def wait_write(r, b):
        pltpu.make_async_copy(
            comp_v.at[b], out_hbm.at[wid * RPW + r], ssems[b]
        ).wait()

    for b in range(NBUF):
        start_gathers(b, b)

    for b in range(NBUF):
        wait_gathers(b, b)
        compact(b)
        start_gathers(b + NBUF, b)
        start_write(b, b)

    def outer(i, _):
        for b in range(NBUF):
            r = i * NBUF + b
            wait_gathers(r, b)
            wait_write(r - NBUF, b)
            compact(b)
            start_gathers(r + NBUF, b)
            start_write(r, b)
        return 0

    lax.fori_loop(1, RPW // NBUF - 1, outer, 0)

    for b in range(NBUF):
        r = (RPW // NBUF - 1) * NBUF + b
        wait_gathers(r, b)
        wait_write(r - NBUF, b)
        compact(b)
        start_write(r, b)

    for b in range(NBUF):
        wait_write((RPW // NBUF - 1) * NBUF + b, b)


def kernel(tokens, table):
    scaled = _scale_table(table)
    out = _emb_lookup(tokens.reshape(-1), scaled)
    return out


# speedup vs baseline: 1.1347x; 1.0541x over previous
import functools

import jax
import jax.numpy as jnp
from jax import lax
from jax.experimental import pallas as pl
from jax.experimental.pallas import tpu as pltpu
from jax.experimental.pallas import tpu_sc as plsc

VOC = 100000
EMB = 64
PAD = 128
SCALE = 8.0

NC = 2
NS = 16
NW = NC * NS

NB = 4096
NT = 200
RPW = NB // NW
NBUF = 2

_mesh = plsc.VectorSubcoreMesh(core_axis_name="c", subcore_axis_name="s")
_params = pltpu.CompilerParams(use_tc_tiling_on_sc=True)


@functools.partial(
    pl.kernel,
    mesh=_mesh,
    out_type=jax.ShapeDtypeStruct((NB, NT, EMB), jnp.float32),
    scratch_types=[
        pltpu.VMEM((RPW * NT,), jnp.int32),
        pltpu.VMEM((NBUF, NT, PAD), jnp.float32),
        pltpu.VMEM((NBUF, NT, EMB), jnp.float32),
        [pltpu.SemaphoreType.DMA] * NBUF,
        [pltpu.SemaphoreType.DMA] * NBUF,
    ],
    compiler_params=_params,
)
def _emb_lookup(tokens_hbm, table_hbm, out_hbm, idx_v, rows_v, comp_v,
                gsems, ssems):
    wid = lax.axis_index("s") * NC + lax.axis_index("c")
    base = wid * RPW * NT

    pltpu.sync_copy(tokens_hbm.at[pl.ds(base, RPW * NT)], idx_v)

    def start_gathers(r, b):
        pltpu.async_copy(
            table_hbm.at[idx_v.at[pl.ds(r * NT, 128)]],
            rows_v.at[b, pl.ds(0, 128)],
            gsems[b],
        )
        pltpu.async_copy(
            table_hbm.at[idx_v.at[pl.ds(r * NT + 128, NT - 128)]],
            rows_v.at[b, pl.ds(128, NT - 128)],
            gsems[b],
        )

    def wait_gathers(r, b):
        pltpu.make_async_copy(
            table_hbm.at[idx_v.at[pl.ds(r * NT, 128)]],
            rows_v.at[b, pl.ds(0, 128)],
            gsems[b],
        ).wait()
        pltpu.make_async_copy(
            table_hbm.at[idx_v.at[pl.ds(r * NT + 128, NT - 128)]],
            rows_v.at[b, pl.ds(128, NT - 128)],
            gsems[b],
        ).wait()

    def scale_compact(b):
        def body(r, _):
            for j in range(EMB // 16):
                comp_v[b, r, pl.ds(j * 16, 16)] = (
                    rows_v[b, r, pl.ds(j * 16, 16)] * SCALE
                )
            return 0

        lax.fori_loop(0, NT, body, 0)

    def start_write(r, b):
        pltpu.async_copy(
            comp_v.at[b], out_hbm.at[wid * RPW + r], ssems[b]
        )

    def wait_write(r, b):
        pltpu.make_async_copy(
            comp_v.at[b], out_hbm.at[wid * RPW + r], ssems[b]
        ).wait()

    for b in range(NBUF):
        start_gathers(b, b)

    for b in range(NBUF):
        wait_gathers(b, b)
        scale_compact(b)
        start_gathers(b + NBUF, b)
        start_write(b, b)

    def outer(i, _):
        for b in range(NBUF):
            r = i * NBUF + b
            wait_gathers(r, b)
            wait_write(r - NBUF, b)
            scale_compact(b)
            start_gathers(r + NBUF, b)
            start_write(r, b)
        return 0

    lax.fori_loop(1, RPW // NBUF - 1, outer, 0)

    for b in range(NBUF):
        r = (RPW // NBUF - 1) * NBUF + b
        wait_gathers(r, b)
        wait_write(r - NBUF, b)
        scale_compact(b)
        start_write(r, b)

    for b in range(NBUF):
        wait_write((RPW // NBUF - 1) * NBUF + b, b)


def kernel(tokens, table):
    table128 = jnp.pad(table, ((0, 0), (0, PAD - EMB)))
    return _emb_lookup(tokens.reshape(-1), table128)


# speedup vs baseline: 1.3551x; 1.1942x over previous
import functools

import jax
import jax.numpy as jnp
from jax import lax
from jax.experimental import pallas as pl
from jax.experimental.pallas import tpu as pltpu
from jax.experimental.pallas import tpu_sc as plsc

VOC = 100000
EMB = 64
PAD = 128
SCALE = 8.0

NC = 2
NS = 16
NW = NC * NS

NB = 4096
NT = 200
TPW = NB * NT // NW
CHUNK = 128
NCH = TPW // CHUNK
NBUF = 2
UNROLL = 8

_mesh = plsc.VectorSubcoreMesh(core_axis_name="c", subcore_axis_name="s")
_params = pltpu.CompilerParams(use_tc_tiling_on_sc=True)


@functools.partial(
    pl.kernel,
    mesh=_mesh,
    out_type=jax.ShapeDtypeStruct((NB * NT, EMB), jnp.float32),
    scratch_types=[
        pltpu.VMEM((TPW,), jnp.int32),
        pltpu.VMEM((NBUF, CHUNK, PAD), jnp.float32),
        pltpu.VMEM((NBUF, CHUNK, EMB), jnp.float32),
        [pltpu.SemaphoreType.DMA] * NBUF,
        [pltpu.SemaphoreType.DMA] * NBUF,
    ],
    compiler_params=_params,
)
def _emb_lookup(tokens_hbm, table_hbm, out_hbm, idx_v, rows_v, comp_v,
                gsems, ssems):
    wid = lax.axis_index("s") * NC + lax.axis_index("c")
    base = wid * TPW

    pltpu.sync_copy(tokens_hbm.at[pl.ds(base, TPW)], idx_v)

    def start_gather(c, b):
        pltpu.async_copy(
            table_hbm.at[idx_v.at[pl.ds(c * CHUNK, CHUNK)]],
            rows_v.at[b],
            gsems[b],
        )

    def wait_gather(c, b):
        pltpu.make_async_copy(
            table_hbm.at[idx_v.at[pl.ds(c * CHUNK, CHUNK)]],
            rows_v.at[b],
            gsems[b],
        ).wait()

    def scale_compact(b):
        def body(g, _):
            r0 = g * UNROLL
            for k in range(UNROLL):
                for j in range(EMB // 16):
                    comp_v[b, r0 + k, pl.ds(j * 16, 16)] = (
                        rows_v[b, r0 + k, pl.ds(j * 16, 16)] * SCALE
                    )
            return 0

        lax.fori_loop(0, CHUNK // UNROLL, body, 0)

    def start_write(c, b):
        pltpu.async_copy(
            comp_v.at[b], out_hbm.at[pl.ds(base + c * CHUNK, CHUNK)], ssems[b]
        )

    def wait_write(c, b):
        pltpu.make_async_copy(
            comp_v.at[b], out_hbm.at[pl.ds(base + c * CHUNK, CHUNK)], ssems[b]
        ).wait()

    for b in range(NBUF):
        start_gather(b, b)

    for b in range(NBUF):
        wait_gather(b, b)
        scale_compact(b)
        start_gather(b + NBUF, b)
        start_write(b, b)

    def outer(i, _):
        for b in range(NBUF):
            c = i * NBUF + b
            wait_gather(c, b)
            wait_write(c - NBUF, b)
            scale_compact(b)
            start_gather(c + NBUF, b)
            start_write(c, b)
        return 0

    lax.fori_loop(1, NCH // NBUF - 1, outer, 0)

    for b in range(NBUF):
        c = (NCH // NBUF - 1) * NBUF + b
        wait_gather(c, b)
        wait_write(c - NBUF, b)
        scale_compact(b)
        start_write(c, b)

    for b in range(NBUF):
        wait_write((NCH // NBUF - 1) * NBUF + b, b)


def kernel(tokens, table):
    table128 = jnp.pad(table, ((0, 0), (0, PAD - EMB)))
    out = _emb_lookup(tokens.reshape(-1), table128)
    return out.reshape(NB, NT, EMB)
